# baseline (speedup 1.0000x reference)
import functools

import jax
import jax.numpy as jnp
from jax import lax
from jax.experimental import pallas as pl
from jax.experimental.pallas import tpu as pltpu
from jax.experimental.pallas import tpu_sc as plsc

N_NODES = 10000
N_EDGES = 320000
FEAT = 128
HF = FEAT // 2

NC = 2
NS = 16
NW = NC * NS
EPW = N_EDGES // NW
CH = 125
NCHUNK = EPW // CH
NBUF = 5
NROUND = NCHUNK // NBUF
NPAD = 10240
RPS = NPAD // NS


@functools.cache
def _make_sc_scatter():
    mesh = plsc.VectorSubcoreMesh(core_axis_name="c", subcore_axis_name="s")

    @functools.partial(
        pl.kernel,
        out_type=(
            jax.ShapeDtypeStruct((NC, NPAD, HF), jnp.float32),
            jax.ShapeDtypeStruct((NC, NPAD, HF), jnp.float32),
        ),
        mesh=mesh,
        compiler_params=pltpu.CompilerParams(use_tc_tiling_on_sc=False),
        scratch_types=(
            [
                pltpu.VMEM((NCHUNK, CH), jnp.int32),
                pltpu.VMEM((NCHUNK, CH), jnp.int32),
                pltpu.VMEM((NBUF, CH, HF), jnp.float32),
                pltpu.VMEM_SHARED((NPAD, HF), jnp.float32),
            ]
            + [pltpu.SemaphoreType.DMA] * (2 * NBUF)
        ),
    )
    def sc_scatter(temp0_hbm, temp1_hbm, src_hbm, dst_hbm, zeros_hbm,
                   out0_hbm, out1_hbm, src_v, dst_v, rows_v, acc, *sems):
        gsem = sems[:NBUF]
        ssem = sems[NBUF:]
        c = lax.axis_index("c")
        s = lax.axis_index("s")
        wid = c * NS + s

        pltpu.sync_copy(src_hbm.at[wid], src_v)
        pltpu.sync_copy(dst_hbm.at[wid], dst_v)

        for h, temp_hbm, out_hbm in ((0, temp0_hbm, out0_hbm),
                                     (1, temp1_hbm, out1_hbm)):
            pltpu.async_copy(zeros_hbm, acc.at[pl.ds(s * RPS, RPS)], ssem[0])

            def gather(i, b):
                return pltpu.async_copy(temp_hbm.at[src_v.at[i]],
                                        rows_v.at[b], gsem[b])

            def gwait(i, b):
                pltpu.make_async_copy(temp_hbm.at[src_v.at[i]],
                                      rows_v.at[b], gsem[b]).wait()

            def scatter(i, b):
                return pltpu.async_copy(rows_v.at[b], acc.at[dst_v.at[i]],
                                        ssem[b], add=True)

            def swait(i, b):
                pltpu.make_async_copy(rows_v.at[b], acc.at[dst_v.at[i]],
                                      ssem[b]).wait()

            for b in range(NBUF):
                gather(b, b)
            pltpu.make_async_copy(zeros_hbm, acc.at[pl.ds(s * RPS, RPS)],
                                  ssem[0]).wait()
            plsc.subcore_barrier()

            def round_body(j, carry):
                i0 = j * NBUF
                for b in range(NBUF):
                    gwait(i0 + b, b)
                    scatter(i0 + b, b)
                for b in range(NBUF):
                    @pl.when(i0 + b + NBUF < NCHUNK)
                    def _refill(b=b, i0=i0):
                        swait(i0 + b, b)
                        gather(i0 + b + NBUF, b)
                return carry

            lax.fori_loop(0, NROUND, round_body, 0)
            for b in range(NBUF):
                swait(NCHUNK - NBUF + b, b)
            plsc.subcore_barrier()
            pltpu.sync_copy(acc.at[pl.ds(s * RPS, RPS)],
                            out_hbm.at[c, pl.ds(s * RPS, RPS)])

    return sc_scatter


def _mm1_body(x_ref, w_ref, b_ref, o0_ref, o1_ref):
    t = (jnp.dot(x_ref[...], w_ref[...], preferred_element_type=jnp.float32)
         + b_ref[...])
    o0_ref[...] = t[:, :HF]
    o1_ref[...] = t[:, HF:]


def _mm2_body(p0_ref, p1_ref, w_ref, b_ref, o0_ref, o1_ref):
    feat0 = p0_ref[0] + p0_ref[1]
    feat1 = p1_ref[0] + p1_ref[1]
    w = w_ref[:FEAT, :] + w_ref[FEAT:, :]
    t = (jnp.dot(feat0, w[:HF, :], preferred_element_type=jnp.float32)
         + jnp.dot(feat1, w[HF:, :], preferred_element_type=jnp.float32)
         + b_ref[...])
    o0_ref[...] = t[:, :HF]
    o1_ref[...] = t[:, HF:]


def _assemble_body(p0_ref, p1_ref, o_ref):
    s0 = p0_ref[0, :N_NODES, :] + p0_ref[1, :N_NODES, :]
    s1 = p1_ref[0, :N_NODES, :] + p1_ref[1, :N_NODES, :]
    o_ref[:, 0 * HF:1 * HF] = s0
    o_ref[:, 1 * HF:2 * HF] = s1
    o_ref[:, 2 * HF:3 * HF] = s0
    o_ref[:, 3 * HF:4 * HF] = s1


def kernel(x, edge_index, W1, b1, W2, b2):
    e32 = edge_index.astype(jnp.int32)
    src = e32[0].reshape(NW, NCHUNK, CH)
    dst = e32[1].reshape(NW, NCHUNK, CH)
    zeros = jnp.zeros((RPS, HF), jnp.float32)
    b1r = b1.reshape(1, FEAT)
    b2r = b2.reshape(1, FEAT)

    half = jax.ShapeDtypeStruct((N_NODES, HF), jnp.float32)
    half_pad = jax.ShapeDtypeStruct((NPAD, HF), jnp.float32)

    t10, t11 = pl.pallas_call(
        _mm1_body, out_shape=(half, half),
    )(x, W1, b1r)

    sc_scatter = _make_sc_scatter()
    p10, p11 = sc_scatter(t10, t11, src, dst, zeros)

    t20, t21 = pl.pallas_call(
        _mm2_body, out_shape=(half_pad, half_pad),
    )(p10, p11, W2, b2r)

    p20, p21 = sc_scatter(t20, t21, src, dst, zeros)

    out = pl.pallas_call(
        _assemble_body,
        out_shape=jax.ShapeDtypeStruct((N_NODES, 2 * FEAT), jnp.float32),
    )(p20, p21)
    return out


# speedup vs baseline: 1.0356x; 1.0356x over previous
import functools

import jax
import jax.numpy as jnp
from jax import lax
from jax.experimental import pallas as pl
from jax.experimental.pallas import tpu as pltpu
from jax.experimental.pallas import tpu_sc as plsc

N_NODES = 10000
N_EDGES = 320000
FEAT = 128
HF = FEAT // 2

NC = 2
NS = 16
NW = NC * NS
EPW = N_EDGES // NW
CH = 125
NCHUNK = EPW // CH
NBUF = 8
NROUND = NCHUNK // NBUF
NPAD = 10240
RPS = NPAD // NS


@functools.cache
def _make_sc_scatter():
    mesh = plsc.VectorSubcoreMesh(core_axis_name="c", subcore_axis_name="s")

    @functools.partial(
        pl.kernel,
        out_type=(
            jax.ShapeDtypeStruct((NC, NPAD, HF), jnp.float32),
            jax.ShapeDtypeStruct((NC, NPAD, HF), jnp.float32),
        ),
        mesh=mesh,
        compiler_params=pltpu.CompilerParams(use_tc_tiling_on_sc=False),
        scratch_types=(
            [
                pltpu.VMEM((NCHUNK, CH), jnp.int32),
                pltpu.VMEM((NCHUNK, CH), jnp.int32),
                pltpu.VMEM((NBUF, CH, HF), jnp.float32),
                pltpu.VMEM_SHARED((NPAD, HF), jnp.float32),
            ]
            + [pltpu.SemaphoreType.DMA] * (2 * NBUF)
        ),
    )
    def sc_scatter(temp0_hbm, temp1_hbm, src_hbm, dst_hbm, zeros_hbm,
                   out0_hbm, out1_hbm, src_v, dst_v, rows_v, acc, *sems):
        gsem = sems[:NBUF]
        ssem = sems[NBUF:]
        c = lax.axis_index("c")
        s = lax.axis_index("s")
        wid = c * NS + s

        pltpu.sync_copy(src_hbm.at[wid], src_v)
        pltpu.sync_copy(dst_hbm.at[wid], dst_v)

        for h, temp_hbm, out_hbm in ((0, temp0_hbm, out0_hbm),
                                     (1, temp1_hbm, out1_hbm)):
            pltpu.async_copy(zeros_hbm, acc.at[pl.ds(s * RPS, RPS)], ssem[0])

            def gather(i, b):
                return pltpu.async_copy(temp_hbm.at[src_v.at[i]],
                                        rows_v.at[b], gsem[b])

            def gwait(i, b):
                pltpu.make_async_copy(temp_hbm.at[src_v.at[i]],
                                      rows_v.at[b], gsem[b]).wait()

            def scatter(i, b):
                return pltpu.async_copy(rows_v.at[b], acc.at[dst_v.at[i]],
                                        ssem[b], add=True)

            def swait(i, b):
                pltpu.make_async_copy(rows_v.at[b], acc.at[dst_v.at[i]],
                                      ssem[b]).wait()

            for b in range(NBUF):
                gather(b, b)
            pltpu.make_async_copy(zeros_hbm, acc.at[pl.ds(s * RPS, RPS)],
                                  ssem[0]).wait()
            plsc.subcore_barrier()

            def round_body(j, carry):
                i0 = j * NBUF
                for b in range(NBUF):
                    gwait(i0 + b, b)
                    scatter(i0 + b, b)
                for b in range(NBUF):
                    @pl.when(i0 + b + NBUF < NCHUNK)
                    def _refill(b=b, i0=i0):
                        swait(i0 + b, b)
                        gather(i0 + b + NBUF, b)
                return carry

            lax.fori_loop(0, NROUND, round_body, 0)
            for b in range(NBUF):
                swait(NCHUNK - NBUF + b, b)
            plsc.subcore_barrier()
            pltpu.sync_copy(acc.at[pl.ds(s * RPS, RPS)],
                            out_hbm.at[c, pl.ds(s * RPS, RPS)])

    return sc_scatter


def _mm1_body(x_ref, w_ref, b_ref, o0_ref, o1_ref):
    t = (jnp.dot(x_ref[...], w_ref[...], preferred_element_type=jnp.float32)
         + b_ref[...])
    o0_ref[...] = t[:, :HF]
    o1_ref[...] = t[:, HF:]


def _mm2_body(p0_ref, p1_ref, w_ref, b_ref, o0_ref, o1_ref):
    feat0 = p0_ref[0] + p0_ref[1]
    feat1 = p1_ref[0] + p1_ref[1]
    w = w_ref[:FEAT, :] + w_ref[FEAT:, :]
    t = (jnp.dot(feat0, w[:HF, :], preferred_element_type=jnp.float32)
         + jnp.dot(feat1, w[HF:, :], preferred_element_type=jnp.float32)
         + b_ref[...])
    o0_ref[...] = t[:, :HF]
    o1_ref[...] = t[:, HF:]


def _assemble_body(p0_ref, p1_ref, o_ref):
    s0 = p0_ref[0, :N_NODES, :] + p0_ref[1, :N_NODES, :]
    s1 = p1_ref[0, :N_NODES, :] + p1_ref[1, :N_NODES, :]
    o_ref[:, 0 * HF:1 * HF] = s0
    o_ref[:, 1 * HF:2 * HF] = s1
    o_ref[:, 2 * HF:3 * HF] = s0
    o_ref[:, 3 * HF:4 * HF] = s1


def kernel(x, edge_index, W1, b1, W2, b2):
    e32 = edge_index.astype(jnp.int32)
    src = e32[0].reshape(NW, NCHUNK, CH)
    dst = e32[1].reshape(NW, NCHUNK, CH)
    zeros = jnp.zeros((RPS, HF), jnp.float32)
    b1r = b1.reshape(1, FEAT)
    b2r = b2.reshape(1, FEAT)

    half = jax.ShapeDtypeStruct((N_NODES, HF), jnp.float32)
    half_pad = jax.ShapeDtypeStruct((NPAD, HF), jnp.float32)

    t10, t11 = pl.pallas_call(
        _mm1_body, out_shape=(half, half),
    )(x, W1, b1r)

    sc_scatter = _make_sc_scatter()
    p10, p11 = sc_scatter(t10, t11, src, dst, zeros)

    t20, t21 = pl.pallas_call(
        _mm2_body, out_shape=(half_pad, half_pad),
    )(p10, p11, W2, b2r)

    p20, p21 = sc_scatter(t20, t21, src, dst, zeros)

    out = pl.pallas_call(
        _assemble_body,
        out_shape=jax.ShapeDtypeStruct((N_NODES, 2 * FEAT), jnp.float32),
    )(p20, p21)
    return out
